# baseline (speedup 1.0000x reference)
import functools

import numpy as np
import jax
import jax.numpy as jnp
from jax import lax
from jax.experimental import pallas as pl
from jax.experimental.pallas import tpu as pltpu
from jax.experimental.pallas import tpu_sc as plsc

C, T, V = 64, 2048, 128
LOW, HIGH = 512, 1024

_rng = np.random.default_rng(0)
_num = int(_rng.integers(LOW, HIGH + 1))
_masked_inds = np.asarray(_rng.choice(T, _num, replace=False), dtype=np.int64)
_fm = np.ones((T,), dtype=np.float32)
_fm[_masked_inds] = 0.0

NC, NS = 2, 16
NW = NC * NS
CPW = C // NW
RPW = CPW * T
ROW_CHUNK = 128
NBUF = 6


def _pad_chunks(a: np.ndarray) -> np.ndarray:
    k = (-len(a)) % ROW_CHUNK
    a = np.concatenate([a, np.repeat(a[-1:], k)])
    return a.reshape(-1, ROW_CHUNK)


def _build_row_lists():
    u_t = np.nonzero(_fm)[0].astype(np.int64)
    m_t = np.nonzero(_fm == 0)[0].astype(np.int64)
    uw, mw = [], []
    for w in range(NW):
        chans = range(w * CPW, (w + 1) * CPW)
        uw.append(_pad_chunks(np.concatenate([c * T + u_t for c in chans])))
        mw.append(_pad_chunks(np.concatenate([c * T + m_t for c in chans])))
    return (np.stack(uw).astype(np.int32), np.stack(mw).astype(np.int32))


_UIDX, _MIDX = _build_row_lists()
UCH = _UIDX.shape[1]
MCH = _MIDX.shape[1]

_mesh = plsc.VectorSubcoreMesh(core_axis_name="c", subcore_axis_name="s")


@functools.partial(
    pl.kernel,
    mesh=_mesh,
    out_type=(
        jax.ShapeDtypeStruct((C * T, V), jnp.float32),
        jax.ShapeDtypeStruct((C * T, V), jnp.float32),
    ),
    scratch_types=[
        pltpu.VMEM((UCH, ROW_CHUNK), jnp.int32),
        pltpu.VMEM((MCH, ROW_CHUNK), jnp.int32),
        pltpu.VMEM((ROW_CHUNK, V), jnp.float32),
        *[pltpu.VMEM((ROW_CHUNK, V), jnp.float32) for _ in range(NBUF)],
        *[pltpu.SemaphoreType.DMA for _ in range(NBUF)],
        *[pltpu.SemaphoreType.DMA for _ in range(NBUF)],
        pltpu.SemaphoreType.DMA,
        pltpu.SemaphoreType.DMA,
    ],
)
def _sc_mask(x_hbm, m_hbm, z_hbm, u_hbm, mi_hbm, xout_hbm, out_hbm,
             uv, mv, zv, *rest):
    bufs = rest[:NBUF]
    gsems = rest[NBUF:2 * NBUF]
    ssems = rest[2 * NBUF:3 * NBUF]
    zsem = rest[3 * NBUF]
    xsem = rest[3 * NBUF + 1]

    wid = lax.axis_index("s") * NC + lax.axis_index("c")
    base = wid * RPW

    xh = pltpu.async_copy(x_hbm.at[pl.ds(base, RPW)],
                          xout_hbm.at[pl.ds(base, RPW)], xsem)

    pltpu.sync_copy(u_hbm.at[wid], uv)
    pltpu.sync_copy(mi_hbm.at[wid], mv)
    pltpu.sync_copy(z_hbm, zv)

    zh = [pltpu.async_copy(zv, out_hbm.at[mv.at[j]], zsem) for j in range(MCH)]

    gh = [None] * UCH
    sh = [None] * UCH
    for j in range(min(NBUF, UCH)):
        gh[j] = pltpu.async_copy(m_hbm.at[uv.at[j]], bufs[j % NBUF], gsems[j % NBUF])
    for j in range(UCH):
        gh[j].wait()
        sh[j] = pltpu.async_copy(bufs[j % NBUF], out_hbm.at[uv.at[j]], ssems[j % NBUF])
        nj = j + NBUF
        if nj < UCH:
            sh[j].wait()
            gh[nj] = pltpu.async_copy(m_hbm.at[uv.at[nj]], bufs[nj % NBUF], gsems[nj % NBUF])
    for j in range(max(0, UCH - NBUF), UCH):
        sh[j].wait()
    for h in zh:
        h.wait()
    xh.wait()


def kernel(x, mask):
    x2d = x.reshape(C * T, V)
    m2d = mask.reshape(C * T, V)
    zeros = jnp.zeros((ROW_CHUNK, V), jnp.float32)
    xout, out = _sc_mask(x2d, m2d, zeros, jnp.asarray(_UIDX), jnp.asarray(_MIDX))
    return (xout.reshape(C, T, V), out.reshape(C, T, V))


# speedup vs baseline: 17.0087x; 17.0087x over previous
import functools

import numpy as np
import jax
import jax.numpy as jnp
from jax import lax
from jax.experimental import pallas as pl
from jax.experimental.pallas import tpu as pltpu
from jax.experimental.pallas import tpu_sc as plsc

C, T, V = 64, 2048, 128
LOW, HIGH = 512, 1024

_rng = np.random.default_rng(0)
_num = int(_rng.integers(LOW, HIGH + 1))
_masked_inds = np.asarray(_rng.choice(T, _num, replace=False), dtype=np.int64)
_fm = np.ones((T,), dtype=np.float32)
_fm[_masked_inds] = 0.0

NC, NS = 2, 16
NW = NC * NS
CPW = C // NW
RPW = CPW * T
ROW_CHUNK = 128
NBUF = 6


def _pad_chunks(a: np.ndarray) -> np.ndarray:
    k = (-len(a)) % ROW_CHUNK
    a = np.concatenate([a, np.repeat(a[-1:], k)])
    return a.reshape(-1, ROW_CHUNK)


def _build_row_lists():
    u_t = np.nonzero(_fm)[0].astype(np.int64)
    m_t = np.nonzero(_fm == 0)[0].astype(np.int64)
    uw, mw = [], []
    for w in range(NW):
        chans = range(w * CPW, (w + 1) * CPW)
        uw.append(_pad_chunks(np.concatenate([c * T + u_t for c in chans])))
        mw.append(_pad_chunks(np.concatenate([c * T + m_t for c in chans])))
    return (np.stack(uw).astype(np.int32), np.stack(mw).astype(np.int32))


_UIDX, _MIDX = _build_row_lists()
UCH = _UIDX.shape[1]
MCH = _MIDX.shape[1]

_mesh = plsc.VectorSubcoreMesh(core_axis_name="c", subcore_axis_name="s")


@functools.partial(
    pl.kernel,
    mesh=_mesh,
    out_type=jax.ShapeDtypeStruct((C * T, V), jnp.float32),
    scratch_types=[
        pltpu.VMEM((UCH, ROW_CHUNK), jnp.int32),
        pltpu.VMEM((MCH, ROW_CHUNK), jnp.int32),
        pltpu.VMEM((ROW_CHUNK, V), jnp.float32),
        *[pltpu.VMEM((ROW_CHUNK, V), jnp.float32) for _ in range(NBUF)],
        *[pltpu.SemaphoreType.DMA for _ in range(NBUF)],
        *[pltpu.SemaphoreType.DMA for _ in range(NBUF)],
        pltpu.SemaphoreType.DMA,
    ],
)
def _sc_mask(m_hbm, z_hbm, u_hbm, mi_hbm, out_hbm, uv, mv, zv, *rest):
    bufs = rest[:NBUF]
    gsems = rest[NBUF:2 * NBUF]
    ssems = rest[2 * NBUF:3 * NBUF]
    zsem = rest[3 * NBUF]

    wid = lax.axis_index("s") * NC + lax.axis_index("c")

    pltpu.sync_copy(u_hbm.at[wid], uv)
    pltpu.sync_copy(mi_hbm.at[wid], mv)
    pltpu.sync_copy(z_hbm, zv)

    zh = [pltpu.async_copy(zv, out_hbm.at[mv.at[j]], zsem) for j in range(MCH)]

    gh = [None] * UCH
    sh = [None] * UCH
    for j in range(min(NBUF, UCH)):
        gh[j] = pltpu.async_copy(m_hbm.at[uv.at[j]], bufs[j % NBUF], gsems[j % NBUF])
    for j in range(UCH):
        gh[j].wait()
        sh[j] = pltpu.async_copy(bufs[j % NBUF], out_hbm.at[uv.at[j]], ssems[j % NBUF])
        nj = j + NBUF
        if nj < UCH:
            sh[j].wait()
            gh[nj] = pltpu.async_copy(m_hbm.at[uv.at[nj]], bufs[nj % NBUF], gsems[nj % NBUF])
    for j in range(max(0, UCH - NBUF), UCH):
        sh[j].wait()
    for h in zh:
        h.wait()


_XBR = 8192


def _copy_body(x_ref, o_ref):
    o_ref[...] = x_ref[...]


def _tc_copy(x2d):
    return pl.pallas_call(
        _copy_body,
        grid=(C * T // _XBR,),
        in_specs=[pl.BlockSpec((_XBR, V), lambda i: (i, 0))],
        out_specs=pl.BlockSpec((_XBR, V), lambda i: (i, 0)),
        out_shape=jax.ShapeDtypeStruct((C * T, V), jnp.float32),
    )(x2d)


def kernel(x, mask):
    x2d = x.reshape(C * T, V)
    m2d = mask.reshape(C * T, V)
    zeros = jnp.zeros((ROW_CHUNK, V), jnp.float32)
    out = _sc_mask(m2d, zeros, jnp.asarray(_UIDX), jnp.asarray(_MIDX))
    xout = _tc_copy(x2d)
    return (xout.reshape(C, T, V), out.reshape(C, T, V))
